# baseline (speedup 1.0000x reference)
import jax
import jax.numpy as jnp
from jax.experimental import pallas as pl
from jax.experimental.pallas import tpu as pltpu

_T = 256
_S = 8


def _mm_kernel(*refs):
    x_ref = refs[0]
    nws = refs[1:1 + _S]
    ows = refs[1 + _S:1 + 2 * _S]
    o1_ref, o2_ref = refs[1 + 2 * _S], refs[2 + 2 * _S]
    w = pl.program_id(0)
    x = x_ref[...]
    kq = x.shape[1] // _S
    xs = [x[:, i * kq:(i + 1) * kq] for i in range(_S)]
    dn = (((1,), (1,)), ((), ()))

    @pl.when(w == 0)
    def _nw():
        o1_ref[...] = sum(
            jax.lax.dot_general(xs[i], nws[i][...], dn,
                                preferred_element_type=jnp.float32)
            for i in range(_S))

    @pl.when(w == 1)
    def _ow():
        o2_ref[...] = sum(
            jax.lax.dot_general(xs[i], ows[i][...], dn,
                                preferred_element_type=jnp.float32)
            for i in range(_S))


def kernel(x, new_weight, orig_weight):
    M, K = x.shape
    N = new_weight.shape[0]
    last = N // _T - 1
    nw_spec = [pl.BlockSpec(
        (_T, K // _S),
        (lambda i: (lambda w, j: (jnp.where(w == 0, j, last), i)))(i))
        for i in range(_S)]
    ow_spec = [pl.BlockSpec(
        (_T, K // _S),
        (lambda i: (lambda w, j: (jnp.where(w == 0, 0, j), i)))(i))
        for i in range(_S)]
    out1, out2 = pl.pallas_call(
        _mm_kernel,
        grid=(2, N // _T),
        in_specs=[pl.BlockSpec((M, K), lambda w, j: (0, 0))] + nw_spec + ow_spec,
        out_specs=[
            pl.BlockSpec((M, _T), lambda w, j: (0, jnp.where(w == 0, j, last))),
            pl.BlockSpec((M, _T), lambda w, j: (0, jnp.where(w == 0, 0, j))),
        ],
        out_shape=[
            jax.ShapeDtypeStruct((M, N), jnp.float32),
            jax.ShapeDtypeStruct((M, N), jnp.float32),
        ],
        compiler_params=pltpu.CompilerParams(
            dimension_semantics=("arbitrary", "arbitrary")),
    )(x, *([new_weight] * _S), *([orig_weight] * _S))
    return (out1, out2)


# speedup vs baseline: 1.1786x; 1.1786x over previous
import jax
import jax.numpy as jnp
from jax.experimental import pallas as pl
from jax.experimental.pallas import tpu as pltpu

_T = 256
_R = 2


def _mm_kernel(*refs):
    x_ref = refs[0]
    nws = refs[1:1 + _R]
    ows = refs[1 + _R:1 + 2 * _R]
    o1_ref, o2_ref = refs[1 + 2 * _R], refs[2 + 2 * _R]
    x = x_ref[...]
    rt = _T // _R
    dn = (((1,), (1,)), ((), ()))
    for i in range(_R):
        o1_ref[:, i * rt:(i + 1) * rt] = jax.lax.dot_general(
            x, nws[i][...], dn, preferred_element_type=jnp.float32)
        o2_ref[:, i * rt:(i + 1) * rt] = jax.lax.dot_general(
            x, ows[i][...], dn, preferred_element_type=jnp.float32)


def kernel(x, new_weight, orig_weight):
    M, K = x.shape
    N = new_weight.shape[0]
    rt = _T // _R
    wspec = [pl.BlockSpec((rt, K), (lambda i: (lambda j: (j * _R + i, 0)))(i))
             for i in range(_R)]
    out1, out2 = pl.pallas_call(
        _mm_kernel,
        grid=(N // _T,),
        in_specs=[pl.BlockSpec((M, K), lambda j: (0, 0))] + wspec + wspec,
        out_specs=[
            pl.BlockSpec((M, _T), lambda j: (0, j)),
            pl.BlockSpec((M, _T), lambda j: (0, j)),
        ],
        out_shape=[
            jax.ShapeDtypeStruct((M, N), jnp.float32),
            jax.ShapeDtypeStruct((M, N), jnp.float32),
        ],
        compiler_params=pltpu.CompilerParams(
            dimension_semantics=("arbitrary",)),
    )(x, *([new_weight] * _R), *([orig_weight] * _R))
    return (out1, out2)
